# baseline (speedup 1.0000x reference)
import functools

import jax
import jax.numpy as jnp
from jax import lax
from jax.experimental import pallas as pl
from jax.experimental.pallas import tpu as pltpu
from jax.experimental.pallas import tpu_sc as plsc

_N = 10000
_D = 128
_E = 320000
_NC = 2
_NT = 16
_K = 128
_RPT = 80
_ROWS = _NC * _NT * _RPT
_EPAD = _ROWS * _K
_NPAD = 10112
_STRIPE = _NPAD // _NT
_KR = 64
_RPTR = _EPAD // (_NC * _NT * _KR)
_ROWSR = _EPAD // _KR
_RUN = 40
_DHALF = 80
_GRP = 4
_LAG = 1
_NPADS = 10240
_STRIPES = _NPADS // _NT
_GRPS = 6
_LAGS = 3
_BLK = 2000
_GRID = _N // _BLK

_mesh = lambda: plsc.VectorSubcoreMesh(core_axis_name="c", subcore_axis_name="s")


def _pipeline(gather_src, rows_v, scatter_dst, gsem, ssem, nchunk, grp, lag):
    gd, sd = {}, {}
    for j in range(nchunk + lag):
        if j < nchunk:
            if j >= grp:
                sd[j - grp].wait()
            gd[j] = pltpu.async_copy(gather_src(j), rows_v.at[j % grp], gsem)
        if j >= lag:
            i = j - lag
            gd[i].wait()
            sd[i] = pltpu.async_copy(rows_v.at[i % grp], scatter_dst(i), ssem,
                                     add=True)
    for i in range(nchunk - grp, nchunk):
        sd[i].wait()


def _agg_rows_body(hs, srcp, dstp, zrows, out, src_v, dst_v, rows_v, acc,
                   gsem, ssem):
    c = lax.axis_index("c")
    s = lax.axis_index("s")
    wid = c * _NT + s
    base = wid * _RPTR
    pltpu.sync_copy(zrows, acc.at[pl.ds(s * _STRIPE, _STRIPE)])
    plsc.subcore_barrier()
    for run in range(_RPTR // _RUN):
        off = run * _RUN
        doff = off % _DHALF
        if doff == 0:
            pltpu.sync_copy(dstp.at[pl.ds(base + off, _DHALF)], dst_v)
        pltpu.sync_copy(srcp.at[pl.ds(base + off, _RUN)], src_v)
        _pipeline(lambda j: hs.at[src_v.at[j]], rows_v,
                  lambda i: acc.at[dst_v.at[doff + i]], gsem, ssem,
                  _RUN, _GRP, _LAG)
    plsc.subcore_barrier()
    pltpu.sync_copy(acc.at[pl.ds(s * _STRIPE, _STRIPE)],
                    out.at[c].at[pl.ds(s * _STRIPE, _STRIPE)])


_agg_rows = functools.partial(
    pl.kernel,
    out_type=jax.ShapeDtypeStruct((_NC, _NPAD, _D), jnp.float32),
    mesh=_mesh(),
    scratch_types=[
        pltpu.VMEM((_RUN, _KR), jnp.int32),
        pltpu.VMEM((_DHALF, _KR), jnp.int32),
        pltpu.VMEM((_GRP, _KR, _D), jnp.float32),
        pltpu.VMEM_SHARED((_NPAD, _D), jnp.float32),
        pltpu.SemaphoreType.DMA,
        pltpu.SemaphoreType.DMA,
    ],
)(_agg_rows_body)


def _deg_body(dstp, zflat, out, dst_v, ones_v, acc, ssem):
    c = lax.axis_index("c")
    s = lax.axis_index("s")
    wid = c * _NT + s
    base = wid * _RPT
    pltpu.sync_copy(dstp.at[pl.ds(base, _RPT)], dst_v)
    for v in range(_K // 16):
        ones_v[pl.ds(v * 16, 16)] = jnp.ones((16,), jnp.float32)
    pltpu.sync_copy(zflat, acc.at[pl.ds(s * _STRIPES, _STRIPES)])
    plsc.subcore_barrier()
    sd = {}
    for j in range(_RPT):
        if j >= _GRPS:
            sd[j - _GRPS].wait()
        sd[j] = pltpu.async_copy(ones_v, acc.at[dst_v.at[j]], ssem, add=True)
    for i in range(_RPT - _GRPS, _RPT):
        sd[i].wait()
    plsc.subcore_barrier()
    pltpu.sync_copy(acc.at[pl.ds(s * _STRIPES, _STRIPES)],
                    out.at[c].at[pl.ds(s * _STRIPES, _STRIPES)])


_deg = functools.partial(
    pl.kernel,
    out_type=jax.ShapeDtypeStruct((_NC, _NPADS), jnp.float32),
    mesh=_mesh(),
    scratch_types=[
        pltpu.VMEM((_RPT, _K), jnp.int32),
        pltpu.VMEM((_K,), jnp.float32),
        pltpu.VMEM_SHARED((_NPADS,), jnp.float32),
        pltpu.SemaphoreType.DMA,
    ],
)(_deg_body)


def _agg_scal_body(vals, srcp, dstp, zflat, out, src_v, dst_v, buf_v, acc,
                   gsem, ssem):
    c = lax.axis_index("c")
    s = lax.axis_index("s")
    wid = c * _NT + s
    base = wid * _RPT
    pltpu.sync_copy(srcp.at[pl.ds(base, _RPT)], src_v)
    pltpu.sync_copy(dstp.at[pl.ds(base, _RPT)], dst_v)
    pltpu.sync_copy(zflat, acc.at[pl.ds(s * _STRIPES, _STRIPES)])
    plsc.subcore_barrier()
    _pipeline(lambda j: vals.at[src_v.at[j]], buf_v,
              lambda i: acc.at[dst_v.at[i]], gsem, ssem, _RPT, _GRPS, _LAGS)
    plsc.subcore_barrier()
    pltpu.sync_copy(acc.at[pl.ds(s * _STRIPES, _STRIPES)],
                    out.at[c].at[pl.ds(s * _STRIPES, _STRIPES)])


_agg_scal = functools.partial(
    pl.kernel,
    out_type=jax.ShapeDtypeStruct((_NC, _NPADS), jnp.float32),
    mesh=_mesh(),
    scratch_types=[
        pltpu.VMEM((_RPT, _K), jnp.int32),
        pltpu.VMEM((_RPT, _K), jnp.int32),
        pltpu.VMEM((_GRPS, _K), jnp.float32),
        pltpu.VMEM_SHARED((_NPADS,), jnp.float32),
        pltpu.SemaphoreType.DMA,
        pltpu.SemaphoreType.DMA,
    ],
)(_agg_scal_body)


def _mm_in_body(x_ref, w_ref, degt_ref, hs_ref):
    dcol = lax.rsqrt(degt_ref[:, 0:1] + degt_ref[:, 1:2] + 1.0)
    h = jnp.dot(x_ref[...], w_ref[...], preferred_element_type=jnp.float32)
    hs_ref[...] = h * dcol


def _mm_in(x, w, degt):
    return pl.pallas_call(
        _mm_in_body,
        grid=(_GRID,),
        in_specs=[
            pl.BlockSpec((_BLK, _D), lambda i: (i, 0)),
            pl.BlockSpec((_D, _D), lambda i: (0, 0)),
            pl.BlockSpec((_BLK, 2), lambda i: (i, 0)),
        ],
        out_specs=pl.BlockSpec((_BLK, _D), lambda i: (i, 0)),
        out_shape=jax.ShapeDtypeStruct((_N, _D), jnp.float32),
    )(x, w, degt)


def _stage_body_res(p0_ref, p1_ref, hs_ref, degt_ref, b_ref, res_ref, w_ref,
                    act_ref, hsn_ref):
    dcol = lax.rsqrt(degt_ref[:, 0:1] + degt_ref[:, 1:2] + 1.0)
    z = dcol * (p0_ref[0] + p1_ref[0] + hs_ref[...]) + b_ref[...]
    z = z + res_ref[...]
    act = jnp.maximum(z, 0.0)
    act_ref[...] = act
    hsn_ref[...] = dcol * jnp.dot(act, w_ref[...],
                                  preferred_element_type=jnp.float32)


def _stage_body_nores(p0_ref, p1_ref, hs_ref, degt_ref, b_ref, w_ref,
                      act_ref, hsn_ref):
    dcol = lax.rsqrt(degt_ref[:, 0:1] + degt_ref[:, 1:2] + 1.0)
    z = dcol * (p0_ref[0] + p1_ref[0] + hs_ref[...]) + b_ref[...]
    act = jnp.maximum(z, 0.0)
    act_ref[...] = act
    hsn_ref[...] = dcol * jnp.dot(act, w_ref[...],
                                  preferred_element_type=jnp.float32)


def _stage(p, hs, degt, b, w, res=None):
    n_in = [p, p, hs, degt, b] + ([res] if res is not None else []) + [w]
    specs = [
        pl.BlockSpec((1, _BLK, _D), lambda i: (0, i, 0)),
        pl.BlockSpec((1, _BLK, _D), lambda i: (1, i, 0)),
        pl.BlockSpec((_BLK, _D), lambda i: (i, 0)),
        pl.BlockSpec((_BLK, 2), lambda i: (i, 0)),
        pl.BlockSpec((1, _D), lambda i: (0, 0)),
    ]
    if res is not None:
        specs.append(pl.BlockSpec((_BLK, _D), lambda i: (i, 0)))
    specs.append(pl.BlockSpec((_D, _D), lambda i: (0, 0)))
    body = _stage_body_res if res is not None else _stage_body_nores
    return pl.pallas_call(
        body,
        grid=(_GRID,),
        in_specs=specs,
        out_specs=[
            pl.BlockSpec((_BLK, _D), lambda i: (i, 0)),
            pl.BlockSpec((_BLK, _D), lambda i: (i, 0)),
        ],
        out_shape=[
            jax.ShapeDtypeStruct((_N, _D), jnp.float32),
            jax.ShapeDtypeStruct((_N, _D), jnp.float32),
        ],
    )(*n_in)


def _final_body(p3t_ref, degt_ref, hsn_ref, bout_ref, o_ref):
    dcol = lax.rsqrt(degt_ref[:, 0:1] + degt_ref[:, 1:2] + 1.0)
    val = dcol * (p3t_ref[:, 0:1] + p3t_ref[:, 1:2] + hsn_ref[:, 0:1])
    o_ref[...] = jnp.broadcast_to(val + bout_ref[0, 0], (_BLK, _D))


def _final(p3t, degt, hsn, bout):
    return pl.pallas_call(
        _final_body,
        grid=(_GRID,),
        in_specs=[
            pl.BlockSpec((_BLK, 2), lambda i: (i, 0)),
            pl.BlockSpec((_BLK, 2), lambda i: (i, 0)),
            pl.BlockSpec((_BLK, _D), lambda i: (i, 0)),
            pl.BlockSpec((1, 1), lambda i: (0, 0)),
        ],
        out_specs=pl.BlockSpec((_BLK, _D), lambda i: (i, 0)),
        out_shape=jax.ShapeDtypeStruct((_N, _D), jnp.float32),
    )(p3t, degt, hsn, bout)


def kernel(x, edge_index, W_in, b_in, W_h, b_h, W_out, b_out):
    f32 = jnp.float32
    src = edge_index[0].astype(jnp.int32)
    dst = edge_index[1].astype(jnp.int32)
    npe = _EPAD - _E
    pad_idx = jnp.arange(npe, dtype=jnp.int32)
    src_p = jnp.concatenate([src, (pad_idx * 53) % _N])
    dst_p = jnp.concatenate([dst, _N + (pad_idx % 16)])
    srcp = src_p.reshape(_ROWS, _K)
    dstp = dst_p.reshape(_ROWS, _K)
    srcpr = src_p.reshape(_ROWSR, _KR)
    dstpr = dst_p.reshape(_ROWSR, _KR)

    zrows = jnp.zeros((_STRIPE, _D), f32)
    zflat = jnp.zeros((_STRIPES,), f32)

    degp = _deg(dstp, zflat)
    degt = degp[:, :_N].T

    hs0 = _mm_in(x, W_in, degt)
    p0 = _agg_rows(hs0, srcpr, dstpr, zrows)
    act0, hs1 = _stage(p0, hs0, degt, b_in.reshape(1, _D), W_h[0])

    p1 = _agg_rows(hs1, srcpr, dstpr, zrows)
    act1, hs2 = _stage(p1, hs1, degt, b_h[0].reshape(1, _D), W_h[1], res=act0)

    w_out_p = jnp.pad(W_out, ((0, 0), (0, _D - W_out.shape[1])))
    p2 = _agg_rows(hs2, srcpr, dstpr, zrows)
    _, hsn2 = _stage(p2, hs2, degt, b_h[1].reshape(1, _D), w_out_p, res=act1)

    p3 = _agg_scal(hsn2[:, 0], srcp, dstp, zflat)
    p3t = p3[:, :_N].T
    wide = _final(p3t, degt, hsn2, b_out.reshape(1, 1))
    return wide[:, :1]


# speedup vs baseline: 1.1035x; 1.1035x over previous
import functools

import jax
import jax.numpy as jnp
from jax import lax
from jax.experimental import pallas as pl
from jax.experimental.pallas import tpu as pltpu
from jax.experimental.pallas import tpu_sc as plsc

_N = 10000
_D = 128
_E = 320000
_NC = 2
_NT = 16
_K = 128
_RPT = 80
_ROWS = _NC * _NT * _RPT
_EPAD = _ROWS * _K
_NPAD = 10112
_STRIPE = _NPAD // _NT
_KR = 64
_RPTR = _EPAD // (_NC * _NT * _KR)
_ROWSR = _EPAD // _KR
_RUN = 40
_DHALF = 80
_GRP = 4
_LAG = 2
_NPADS = 10240
_STRIPES = _NPADS // _NT
_GRPS = 6
_LAGS = 3
_BLK = 2000
_GRID = _N // _BLK

_mesh = lambda: plsc.VectorSubcoreMesh(core_axis_name="c", subcore_axis_name="s")


def _pipeline(gather_src, rows_v, scatter_dst, gsem, ssem, nchunk, grp, lag):
    gd, sd = {}, {}
    for j in range(nchunk + lag):
        if j < nchunk:
            if j >= grp:
                sd[j - grp].wait()
            gd[j] = pltpu.async_copy(gather_src(j), rows_v.at[j % grp], gsem)
        if j >= lag:
            i = j - lag
            gd[i].wait()
            sd[i] = pltpu.async_copy(rows_v.at[i % grp], scatter_dst(i), ssem,
                                     add=True)
    for i in range(nchunk - grp, nchunk):
        sd[i].wait()


def _agg_rows_body(hs, srcp, dstp, zrows, out, src_v, dst_v, rows_v, acc,
                   gsem, ssem):
    c = lax.axis_index("c")
    s = lax.axis_index("s")
    wid = c * _NT + s
    base = wid * _RPTR
    pltpu.sync_copy(zrows, acc.at[pl.ds(s * _STRIPE, _STRIPE)])
    plsc.subcore_barrier()
    for run in range(_RPTR // _RUN):
        off = run * _RUN
        doff = off % _DHALF
        if doff == 0:
            pltpu.sync_copy(dstp.at[pl.ds(base + off, _DHALF)], dst_v)
        pltpu.sync_copy(srcp.at[pl.ds(base + off, _RUN)], src_v)
        _pipeline(lambda j: hs.at[src_v.at[j]], rows_v,
                  lambda i: acc.at[dst_v.at[doff + i]], gsem, ssem,
                  _RUN, _GRP, _LAG)
    plsc.subcore_barrier()
    pltpu.sync_copy(acc.at[pl.ds(s * _STRIPE, _STRIPE)],
                    out.at[c].at[pl.ds(s * _STRIPE, _STRIPE)])


_agg_rows = functools.partial(
    pl.kernel,
    out_type=jax.ShapeDtypeStruct((_NC, _NPAD, _D), jnp.float32),
    mesh=_mesh(),
    scratch_types=[
        pltpu.VMEM((_RUN, _KR), jnp.int32),
        pltpu.VMEM((_DHALF, _KR), jnp.int32),
        pltpu.VMEM((_GRP, _KR, _D), jnp.float32),
        pltpu.VMEM_SHARED((_NPAD, _D), jnp.float32),
        pltpu.SemaphoreType.DMA,
        pltpu.SemaphoreType.DMA,
    ],
)(_agg_rows_body)


def _deg_body(dstp, zflat, out, dst_v, ones_v, acc, ssem):
    c = lax.axis_index("c")
    s = lax.axis_index("s")
    wid = c * _NT + s
    base = wid * _RPT
    pltpu.sync_copy(dstp.at[pl.ds(base, _RPT)], dst_v)
    for v in range(_K // 16):
        ones_v[pl.ds(v * 16, 16)] = jnp.ones((16,), jnp.float32)
    pltpu.sync_copy(zflat, acc.at[pl.ds(s * _STRIPES, _STRIPES)])
    plsc.subcore_barrier()
    sd = {}
    for j in range(_RPT):
        if j >= _GRPS:
            sd[j - _GRPS].wait()
        sd[j] = pltpu.async_copy(ones_v, acc.at[dst_v.at[j]], ssem, add=True)
    for i in range(_RPT - _GRPS, _RPT):
        sd[i].wait()
    plsc.subcore_barrier()
    pltpu.sync_copy(acc.at[pl.ds(s * _STRIPES, _STRIPES)],
                    out.at[c].at[pl.ds(s * _STRIPES, _STRIPES)])


_deg = functools.partial(
    pl.kernel,
    out_type=jax.ShapeDtypeStruct((_NC, _NPADS), jnp.float32),
    mesh=_mesh(),
    scratch_types=[
        pltpu.VMEM((_RPT, _K), jnp.int32),
        pltpu.VMEM((_K,), jnp.float32),
        pltpu.VMEM_SHARED((_NPADS,), jnp.float32),
        pltpu.SemaphoreType.DMA,
    ],
)(_deg_body)


def _agg_scal_body(vals, srcp, dstp, zflat, out, src_v, dst_v, buf_v, acc,
                   gsem, ssem):
    c = lax.axis_index("c")
    s = lax.axis_index("s")
    wid = c * _NT + s
    base = wid * _RPT
    pltpu.sync_copy(srcp.at[pl.ds(base, _RPT)], src_v)
    pltpu.sync_copy(dstp.at[pl.ds(base, _RPT)], dst_v)
    pltpu.sync_copy(zflat, acc.at[pl.ds(s * _STRIPES, _STRIPES)])
    plsc.subcore_barrier()
    _pipeline(lambda j: vals.at[src_v.at[j]], buf_v,
              lambda i: acc.at[dst_v.at[i]], gsem, ssem, _RPT, _GRPS, _LAGS)
    plsc.subcore_barrier()
    pltpu.sync_copy(acc.at[pl.ds(s * _STRIPES, _STRIPES)],
                    out.at[c].at[pl.ds(s * _STRIPES, _STRIPES)])


_agg_scal = functools.partial(
    pl.kernel,
    out_type=jax.ShapeDtypeStruct((_NC, _NPADS), jnp.float32),
    mesh=_mesh(),
    scratch_types=[
        pltpu.VMEM((_RPT, _K), jnp.int32),
        pltpu.VMEM((_RPT, _K), jnp.int32),
        pltpu.VMEM((_GRPS, _K), jnp.float32),
        pltpu.VMEM_SHARED((_NPADS,), jnp.float32),
        pltpu.SemaphoreType.DMA,
        pltpu.SemaphoreType.DMA,
    ],
)(_agg_scal_body)


def _mm_in_body(x_ref, w_ref, degt_ref, hs_ref):
    dcol = lax.rsqrt(degt_ref[:, 0:1] + degt_ref[:, 1:2] + 1.0)
    h = jnp.dot(x_ref[...], w_ref[...], preferred_element_type=jnp.float32)
    hs_ref[...] = h * dcol


def _mm_in(x, w, degt):
    return pl.pallas_call(
        _mm_in_body,
        grid=(_GRID,),
        in_specs=[
            pl.BlockSpec((_BLK, _D), lambda i: (i, 0)),
            pl.BlockSpec((_D, _D), lambda i: (0, 0)),
            pl.BlockSpec((_BLK, 2), lambda i: (i, 0)),
        ],
        out_specs=pl.BlockSpec((_BLK, _D), lambda i: (i, 0)),
        out_shape=jax.ShapeDtypeStruct((_N, _D), jnp.float32),
    )(x, w, degt)


def _stage_body_res(p0_ref, p1_ref, hs_ref, degt_ref, b_ref, res_ref, w_ref,
                    act_ref, hsn_ref):
    dcol = lax.rsqrt(degt_ref[:, 0:1] + degt_ref[:, 1:2] + 1.0)
    z = dcol * (p0_ref[0] + p1_ref[0] + hs_ref[...]) + b_ref[...]
    z = z + res_ref[...]
    act = jnp.maximum(z, 0.0)
    act_ref[...] = act
    hsn_ref[...] = dcol * jnp.dot(act, w_ref[...],
                                  preferred_element_type=jnp.float32)


def _stage_body_nores(p0_ref, p1_ref, hs_ref, degt_ref, b_ref, w_ref,
                      act_ref, hsn_ref):
    dcol = lax.rsqrt(degt_ref[:, 0:1] + degt_ref[:, 1:2] + 1.0)
    z = dcol * (p0_ref[0] + p1_ref[0] + hs_ref[...]) + b_ref[...]
    act = jnp.maximum(z, 0.0)
    act_ref[...] = act
    hsn_ref[...] = dcol * jnp.dot(act, w_ref[...],
                                  preferred_element_type=jnp.float32)


def _stage(p, hs, degt, b, w, res=None):
    n_in = [p, p, hs, degt, b] + ([res] if res is not None else []) + [w]
    specs = [
        pl.BlockSpec((1, _BLK, _D), lambda i: (0, i, 0)),
        pl.BlockSpec((1, _BLK, _D), lambda i: (1, i, 0)),
        pl.BlockSpec((_BLK, _D), lambda i: (i, 0)),
        pl.BlockSpec((_BLK, 2), lambda i: (i, 0)),
        pl.BlockSpec((1, _D), lambda i: (0, 0)),
    ]
    if res is not None:
        specs.append(pl.BlockSpec((_BLK, _D), lambda i: (i, 0)))
    specs.append(pl.BlockSpec((_D, _D), lambda i: (0, 0)))
    body = _stage_body_res if res is not None else _stage_body_nores
    return pl.pallas_call(
        body,
        grid=(_GRID,),
        in_specs=specs,
        out_specs=[
            pl.BlockSpec((_BLK, _D), lambda i: (i, 0)),
            pl.BlockSpec((_BLK, _D), lambda i: (i, 0)),
        ],
        out_shape=[
            jax.ShapeDtypeStruct((_N, _D), jnp.float32),
            jax.ShapeDtypeStruct((_N, _D), jnp.float32),
        ],
    )(*n_in)


def _final_body(p3t_ref, degt_ref, hsn_ref, bout_ref, o_ref):
    dcol = lax.rsqrt(degt_ref[:, 0:1] + degt_ref[:, 1:2] + 1.0)
    val = dcol * (p3t_ref[:, 0:1] + p3t_ref[:, 1:2] + hsn_ref[:, 0:1])
    o_ref[...] = jnp.broadcast_to(val + bout_ref[0, 0], (_BLK, _D))


def _final(p3t, degt, hsn, bout):
    return pl.pallas_call(
        _final_body,
        grid=(_GRID,),
        in_specs=[
            pl.BlockSpec((_BLK, 2), lambda i: (i, 0)),
            pl.BlockSpec((_BLK, 2), lambda i: (i, 0)),
            pl.BlockSpec((_BLK, _D), lambda i: (i, 0)),
            pl.BlockSpec((1, 1), lambda i: (0, 0)),
        ],
        out_specs=pl.BlockSpec((_BLK, _D), lambda i: (i, 0)),
        out_shape=jax.ShapeDtypeStruct((_N, _D), jnp.float32),
    )(p3t, degt, hsn, bout)


def kernel(x, edge_index, W_in, b_in, W_h, b_h, W_out, b_out):
    f32 = jnp.float32
    src = edge_index[0].astype(jnp.int32)
    dst = edge_index[1].astype(jnp.int32)
    npe = _EPAD - _E
    pad_idx = jnp.arange(npe, dtype=jnp.int32)
    src_p = jnp.concatenate([src, (pad_idx * 53) % _N])
    dst_p = jnp.concatenate([dst, _N + (pad_idx % 16)])
    srcp = src_p.reshape(_ROWS, _K)
    dstp = dst_p.reshape(_ROWS, _K)
    srcpr = src_p.reshape(_ROWSR, _KR)
    dstpr = dst_p.reshape(_ROWSR, _KR)

    zrows = jnp.zeros((_STRIPE, _D), f32)
    zflat = jnp.zeros((_STRIPES,), f32)

    degp = _deg(dstp, zflat)
    degt = degp[:, :_N].T

    hs0 = _mm_in(x, W_in, degt)
    p0 = _agg_rows(hs0, srcpr, dstpr, zrows)
    act0, hs1 = _stage(p0, hs0, degt, b_in.reshape(1, _D), W_h[0])

    p1 = _agg_rows(hs1, srcpr, dstpr, zrows)
    act1, hs2 = _stage(p1, hs1, degt, b_h[0].reshape(1, _D), W_h[1], res=act0)

    w_out_p = jnp.pad(W_out, ((0, 0), (0, _D - W_out.shape[1])))
    p2 = _agg_rows(hs2, srcpr, dstpr, zrows)
    _, hsn2 = _stage(p2, hs2, degt, b_h[1].reshape(1, _D), w_out_p, res=act1)

    p3 = _agg_scal(hsn2[:, 0], srcp, dstp, zflat)
    p3t = p3[:, :_N].T
    wide = _final(p3t, degt, hsn2, b_out.reshape(1, 1))
    return wide[:, :1]
